# baseline (speedup 1.0000x reference)
import jax
import jax.numpy as jnp
from jax.experimental import pallas as pl
from jax.experimental.pallas import tpu as pltpu

TOK_BLK = 1024


def _router_kernel(x_ref, w_ref, o_ref):
    x = x_ref[...].astype(jnp.bfloat16)
    w = w_ref[...].astype(jnp.bfloat16)
    o_ref[...] = jax.lax.dot_general(
        x, w,
        dimension_numbers=(((1,), (1,)), ((), ())),
        preferred_element_type=jnp.float32)


def kernel(hidden_states, W):
    B, S, H = hidden_states.shape
    E = W.shape[0]
    T = B * S
    x = hidden_states.reshape(T, H)
    out = pl.pallas_call(
        _router_kernel,
        grid=(T // TOK_BLK,),
        in_specs=[
            pl.BlockSpec((TOK_BLK, H), lambda i: (i, 0)),
            pl.BlockSpec((E, H), lambda i: (0, 0)),
        ],
        out_specs=pl.BlockSpec((TOK_BLK, E), lambda i: (i, 0)),
        out_shape=jax.ShapeDtypeStruct((T, E), jnp.float32),
        compiler_params=pltpu.CompilerParams(
            dimension_semantics=("arbitrary",),
        ),
    )(x, W)
    return out.reshape(B, S, E)
